# Initial kernel scaffold; baseline (speedup 1.0000x reference)
import jax
import jax.numpy as jnp
from jax.experimental import pallas as pl


def kernel(in_feat, edge_index, W1, b1, W2, b2):
    raise NotImplementedError("write your pallas kernel here")


# speedup vs baseline: 5.5272x; 5.5272x over previous
import functools

import jax
import jax.numpy as jnp
from jax import lax
from jax.experimental import pallas as pl
from jax.experimental.pallas import tpu as pltpu
from jax.experimental.pallas import tpu_sc as plsc

N = 10000
E = 320000
D = 128

NC = 2
NS = 16
NW = NC * NS
B = 128
NB = 79
EPT = NB * B
EPAD = NW * EPT
NPADROWS = 240
N1 = N + NPADROWS
SLICE = N1 // NS
BLK = 1024

_mesh = plsc.VectorSubcoreMesh(core_axis_name="c", subcore_axis_name="s")


def _sc_degrees(src_pad, dst_pad):
  @functools.partial(
      pl.kernel,
      out_type=jax.ShapeDtypeStruct((NC * 2 * N1,), jnp.float32),
      mesh=_mesh,
      scratch_types=[
          pltpu.VMEM((B,), jnp.int32),
          pltpu.VMEM((B,), jnp.int32),
          pltpu.VMEM((B,), jnp.float32),
          pltpu.VMEM((SLICE,), jnp.float32),
          pltpu.VMEM_SHARED((N1,), jnp.float32),
          pltpu.VMEM_SHARED((N1,), jnp.float32),
      ],
  )
  def k(src_hbm, dst_hbm, out_hbm, src_v, dst_v, ones_v, stage_v, dsrc_sh,
        ddst_sh):
    cid = lax.axis_index("c")
    sid = lax.axis_index("s")
    wid = cid * NS + sid
    one16 = jnp.ones((16,), jnp.float32)
    zero16 = jnp.zeros((16,), jnp.float32)
    for jj in range(B // 16):
      ones_v[pl.ds(jj * 16, 16)] = one16

    def fill_zero(r, carry):
      stage_v[pl.ds(r * 16, 16)] = zero16
      return carry

    lax.fori_loop(0, SLICE // 16, fill_zero, 0)
    row0 = sid * SLICE
    pltpu.sync_copy(stage_v, dsrc_sh.at[pl.ds(row0, SLICE)])
    pltpu.sync_copy(stage_v, ddst_sh.at[pl.ds(row0, SLICE)])
    plsc.subcore_barrier()

    def body(b, carry):
      off = wid * EPT + b * B
      pltpu.sync_copy(src_hbm.at[pl.ds(off, B)], src_v)
      pltpu.sync_copy(dst_hbm.at[pl.ds(off, B)], dst_v)
      pltpu.sync_copy(ones_v, dsrc_sh.at[src_v], add=True)
      pltpu.sync_copy(ones_v, ddst_sh.at[dst_v], add=True)
      return carry

    lax.fori_loop(0, NB, body, 0)
    plsc.subcore_barrier()
    pltpu.sync_copy(dsrc_sh.at[pl.ds(row0, SLICE)],
                    out_hbm.at[pl.ds(cid * 2 * N1 + row0, SLICE)])
    pltpu.sync_copy(ddst_sh.at[pl.ds(row0, SLICE)],
                    out_hbm.at[pl.ds(cid * 2 * N1 + N1 + row0, SLICE)])

  return k(src_pad, dst_pad).reshape(NC, 2, N1)


def _sc_aggregate(h, src_pad, dst_pad, zrows):
  @functools.partial(
      pl.kernel,
      out_type=jax.ShapeDtypeStruct((NC * N1, D), jnp.float32),
      mesh=_mesh,
      scratch_types=[
          pltpu.VMEM((B,), jnp.int32),
          pltpu.VMEM((B,), jnp.int32),
          pltpu.VMEM((B, D), jnp.float32),
          pltpu.VMEM_SHARED((N1, D), jnp.float32),
          pltpu.SemaphoreType.DMA,
      ],
  )
  def k(h_hbm, src_hbm, dst_hbm, z_hbm, out_hbm, src_v, dst_v, rows_v, acc_sh,
        sem):
    cid = lax.axis_index("c")
    sid = lax.axis_index("s")
    wid = cid * NS + sid
    row0 = sid * SLICE
    pltpu.sync_copy(z_hbm, acc_sh.at[pl.ds(row0, SLICE)])
    plsc.subcore_barrier()

    def body(b, carry):
      off = wid * EPT + b * B
      pltpu.sync_copy(src_hbm.at[pl.ds(off, B)], src_v)
      pltpu.async_copy(h_hbm.at[src_v], rows_v, sem).wait()
      pltpu.sync_copy(dst_hbm.at[pl.ds(off, B)], dst_v)
      pltpu.sync_copy(rows_v, acc_sh.at[dst_v], add=True)
      return carry

    lax.fori_loop(0, NB, body, 0)
    plsc.subcore_barrier()
    pltpu.sync_copy(acc_sh.at[pl.ds(row0, SLICE)],
                    out_hbm.at[pl.ds(cid * N1 + row0, SLICE)])

  return k(h, src_pad, dst_pad, zrows).reshape(NC, N1, D)


def _tc_prep(deg, x):
  grid = N1 // BLK

  def body(deg_ref, x_ref, h0_ref, dout_ref, din_ref):
    dsrc = deg_ref[0, 0] + deg_ref[1, 0]
    ddst = deg_ref[0, 1] + deg_ref[1, 1]
    dout = lax.rsqrt(jnp.maximum(dsrc, 1.0))
    din = lax.rsqrt(jnp.maximum(ddst, 1.0))
    dout_ref[...] = dout
    din_ref[...] = din
    h0_ref[...] = x_ref[...] * dout[:, None]

  return pl.pallas_call(
      body,
      grid=(grid,),
      in_specs=[
          pl.BlockSpec((NC, 2, BLK), lambda i: (0, 0, i)),
          pl.BlockSpec((BLK, D), lambda i: (i, 0)),
      ],
      out_specs=[
          pl.BlockSpec((BLK, D), lambda i: (i, 0)),
          pl.BlockSpec((BLK,), lambda i: (i,)),
          pl.BlockSpec((BLK,), lambda i: (i,)),
      ],
      out_shape=[
          jax.ShapeDtypeStruct((N1, D), jnp.float32),
          jax.ShapeDtypeStruct((N1,), jnp.float32),
          jax.ShapeDtypeStruct((N1,), jnp.float32),
      ],
  )(deg, x)


def _tc_layer(p, din, dout, W, b2d, relu_scale):
  grid = N1 // BLK

  def body(p_ref, din_ref, dout_ref, w_ref, b_ref, out_ref):
    agg = (p_ref[0] + p_ref[1]) * din_ref[...][:, None]
    y = jnp.dot(agg, w_ref[...], preferred_element_type=jnp.float32)
    y = y + b_ref[...]
    if relu_scale:
      y = jnp.maximum(y, 0.0) * dout_ref[...][:, None]
    out_ref[...] = y

  return pl.pallas_call(
      body,
      grid=(grid,),
      in_specs=[
          pl.BlockSpec((NC, BLK, D), lambda i: (0, i, 0)),
          pl.BlockSpec((BLK,), lambda i: (i,)),
          pl.BlockSpec((BLK,), lambda i: (i,)),
          pl.BlockSpec((D, D), lambda i: (0, 0)),
          pl.BlockSpec((1, D), lambda i: (0, 0)),
      ],
      out_specs=pl.BlockSpec((BLK, D), lambda i: (i, 0)),
      out_shape=jax.ShapeDtypeStruct((N1, D), jnp.float32),
  )(p, din, dout, W, b2d)


@jax.jit
def kernel(in_feat, edge_index, W1, b1, W2, b2):
  src = edge_index[0]
  dst = edge_index[1]
  npad = EPAD - E
  pad_ids = jnp.arange(npad, dtype=jnp.int32)
  pad_rows = N + pad_ids % NPADROWS
  src_agg = jnp.concatenate([src, pad_ids % N])
  src_deg = jnp.concatenate([src, pad_rows])
  dst_pad = jnp.concatenate([dst, pad_rows])
  x_pad = jnp.concatenate([in_feat, jnp.zeros((N1 - N, D), jnp.float32)])
  zrows = jnp.zeros((SLICE, D), jnp.float32)

  deg = _sc_degrees(src_deg, dst_pad)
  h0, dout, din = _tc_prep(deg, x_pad)
  p1 = _sc_aggregate(h0[:N], src_agg, dst_pad, zrows)
  h1s = _tc_layer(p1, din, dout, W1, b1.reshape(1, D), True)
  p2 = _sc_aggregate(h1s[:N], src_agg, dst_pad, zrows)
  out = _tc_layer(p2, din, dout, W2, b2.reshape(1, D), False)
  return out[:N]
